# Initial kernel scaffold; baseline (speedup 1.0000x reference)
import functools

import jax
import jax.numpy as jnp
from jax import lax
from jax.experimental import pallas as pl
from jax.experimental.pallas import tpu as pltpu
from jax.experimental.pallas import tpu_sc as plsc

N = 10000
D = 128
E = 320000
NC = 2
NS = 16
L = 16
NW = NC * NS
C = 128
CH = 80
EPT = C * CH
EPAD = NW * EPT
NPR = 10240
RPT = NPR // NS

_mesh = plsc.VectorSubcoreMesh(core_axis_name="c", subcore_axis_name="s")


@functools.partial(
    pl.kernel,
    out_type=(
        jax.ShapeDtypeStruct((NC, NPR, L), jnp.float32),
        jax.ShapeDtypeStruct((NC, NPR, L), jnp.float32),
    ),
    mesh=_mesh,
    scratch_types=[
        pltpu.VMEM((CH, C), jnp.int32),
        pltpu.VMEM((CH, C), jnp.int32),
        pltpu.VMEM((C, L), jnp.float32),
        pltpu.VMEM((RPT, L), jnp.float32),
        pltpu.VMEM_SHARED((NPR, L), jnp.float32),
        pltpu.VMEM_SHARED((NPR, L), jnp.float32),
    ],
)
def _deg_kernel(src3, dst3, dego, degi, idxs, idxd, ones_v, zer_v, sh_o, sh_i):
    c = lax.axis_index("c")
    s = lax.axis_index("s")
    wid = c * NS + s

    def fill_ones(i, carry):
        ones_v[i, :] = jnp.full((L,), 1.0, jnp.float32)
        return carry

    lax.fori_loop(0, C, fill_ones, 0)

    def fill_zeros(i, carry):
        zer_v[i, :] = jnp.zeros((L,), jnp.float32)
        return carry

    lax.fori_loop(0, RPT, fill_zeros, 0)

    pltpu.sync_copy(zer_v, sh_o.at[pl.ds(s * RPT, RPT)])
    pltpu.sync_copy(zer_v, sh_i.at[pl.ds(s * RPT, RPT)])
    pltpu.sync_copy(src3.at[wid], idxs)
    pltpu.sync_copy(dst3.at[wid], idxd)
    plsc.subcore_barrier()

    def chunk(j, carry):
        pltpu.sync_copy(ones_v, sh_o.at[idxs.at[j]], add=True)
        pltpu.sync_copy(ones_v, sh_i.at[idxd.at[j]], add=True)
        return carry

    lax.fori_loop(0, CH, chunk, 0)
    plsc.subcore_barrier()

    pltpu.sync_copy(sh_o.at[pl.ds(s * RPT, RPT)], dego.at[c, pl.ds(s * RPT, RPT)])
    pltpu.sync_copy(sh_i.at[pl.ds(s * RPT, RPT)], degi.at[c, pl.ds(s * RPT, RPT)])


@functools.partial(
    pl.kernel,
    out_type=jax.ShapeDtypeStruct((NC, NPR, D), jnp.float32),
    mesh=_mesh,
    scratch_types=[
        pltpu.VMEM((CH, C), jnp.int32),
        pltpu.VMEM((CH, C), jnp.int32),
        pltpu.VMEM((C, D), jnp.float32),
        pltpu.VMEM((C, D), jnp.float32),
        pltpu.VMEM((C, D), jnp.float32),
        pltpu.VMEM_SHARED((NPR, D), jnp.float32),
        pltpu.SemaphoreType.DMA,
        pltpu.SemaphoreType.DMA,
    ],
)
def _agg_kernel(h_hbm, src3, dst3, aggp, idxs, idxd, vb0, vb1, zer_v, sh_a,
                sem0, sem1):
    c = lax.axis_index("c")
    s = lax.axis_index("s")
    wid = c * NS + s

    def fill_zeros(i, carry):
        for g in range(D // L):
            zer_v[i, pl.ds(g * L, L)] = jnp.zeros((L,), jnp.float32)
        return carry

    lax.fori_loop(0, C, fill_zeros, 0)
    for k in range(RPT // C):
        pltpu.sync_copy(zer_v, sh_a.at[pl.ds(s * RPT + k * C, C)])

    pltpu.sync_copy(src3.at[wid], idxs)
    pltpu.sync_copy(dst3.at[wid], idxd)
    plsc.subcore_barrier()

    pltpu.async_copy(h_hbm.at[idxs.at[0]], vb0, sem0)
    pltpu.async_copy(h_hbm.at[idxs.at[1]], vb1, sem1)

    def step(jj, carry):
        j = jj * 2
        pltpu.make_async_copy(h_hbm.at[idxs.at[j]], vb0, sem0).wait()
        pltpu.sync_copy(vb0, sh_a.at[idxd.at[j]], add=True)
        pltpu.async_copy(h_hbm.at[idxs.at[j + 2]], vb0, sem0)
        pltpu.make_async_copy(h_hbm.at[idxs.at[j + 1]], vb1, sem1).wait()
        pltpu.sync_copy(vb1, sh_a.at[idxd.at[j + 1]], add=True)
        pltpu.async_copy(h_hbm.at[idxs.at[j + 3]], vb1, sem1)
        return carry

    lax.fori_loop(0, CH // 2 - 1, step, 0)
    jt = CH - 2
    pltpu.make_async_copy(h_hbm.at[idxs.at[jt]], vb0, sem0).wait()
    pltpu.sync_copy(vb0, sh_a.at[idxd.at[jt]], add=True)
    pltpu.make_async_copy(h_hbm.at[idxs.at[jt + 1]], vb1, sem1).wait()
    pltpu.sync_copy(vb1, sh_a.at[idxd.at[jt + 1]], add=True)

    plsc.subcore_barrier()
    pltpu.sync_copy(sh_a.at[pl.ds(s * RPT, RPT)], aggp.at[c, pl.ds(s * RPT, RPT)])


def _scale_body(dego_ref, x_ref, h_ref):
    d = dego_ref[0] + dego_ref[1]
    nrm = lax.rsqrt(jnp.maximum(d[:, :1], 1.0))
    h_ref[...] = x_ref[...] * nrm


_scale = pl.pallas_call(
    _scale_body,
    out_shape=jax.ShapeDtypeStruct((NPR, D), jnp.float32),
)


def _out_body(degi_ref, agg_ref, x_ref, w_ref, b_ref, o_ref):
    d = degi_ref[0] + degi_ref[1]
    nrm = lax.rsqrt(jnp.maximum(d[:, :1], 1.0))
    a = (agg_ref[0] + agg_ref[1]) * nrm
    a = a[:N]
    o_ref[...] = (
        jnp.dot(a, w_ref[...], preferred_element_type=jnp.float32)
        + b_ref[...][None, :]
        + x_ref[...]
    )


_finish = pl.pallas_call(
    _out_body,
    out_shape=jax.ShapeDtypeStruct((N, D), jnp.float32),
)


def kernel(x, edge_index, W, b):
    src = edge_index[0].astype(jnp.int32)
    dst = edge_index[1].astype(jnp.int32)
    pad = jnp.full((EPAD - E,), N, jnp.int32)
    src3 = jnp.concatenate([src, pad]).reshape(NW, CH, C)
    dst3 = jnp.concatenate([dst, pad]).reshape(NW, CH, C)
    x_pad = jnp.concatenate([x, jnp.zeros((NPR - N, D), jnp.float32)])

    dego, degi = _deg_kernel(src3, dst3)
    h_pad = _scale(dego, x_pad)
    aggp = _agg_kernel(h_pad, src3, dst3)
    out = _finish(degi, aggp, x, W, b)
    return out


# speedup vs baseline: 4.9393x; 4.9393x over previous
import functools

import jax
import jax.numpy as jnp
from jax import lax
from jax.experimental import pallas as pl
from jax.experimental.pallas import tpu as pltpu
from jax.experimental.pallas import tpu_sc as plsc

N = 10000
D = 128
E = 320000
NC = 2
NS = 16
L = 16
NW = NC * NS
C = 128
CH = 80
PH = 2
CP = CH // PH
EPT = C * CH
EPAD = NW * EPT
NPR = 10240
RPT = NPR // NS

_mesh = plsc.VectorSubcoreMesh(core_axis_name="c", subcore_axis_name="s")


@functools.partial(
    pl.kernel,
    out_type=(
        jax.ShapeDtypeStruct((NC, NPR), jnp.float32),
        jax.ShapeDtypeStruct((NC, NPR), jnp.float32),
    ),
    mesh=_mesh,
    compiler_params=pltpu.CompilerParams(needs_layout_passes=False),
    scratch_types=[
        pltpu.VMEM((CH, C), jnp.int32),
        pltpu.VMEM((CH, C), jnp.int32),
        pltpu.VMEM((NPR,), jnp.float32),
        pltpu.VMEM((NPR,), jnp.float32),
        pltpu.VMEM((NS, RPT), jnp.float32),
        pltpu.VMEM((RPT,), jnp.float32),
        pltpu.VMEM_SHARED((NS, NPR), jnp.float32),
    ],
)
def _deg_kernel(src3, dst3, dego, degi, idxs, idxd, ho, hi, tmp, res, sh):
    c = lax.axis_index("c")
    s = lax.axis_index("s")
    wid = c * NS + s

    def z(i, carry):
        ho[pl.ds(i * L, L)] = jnp.zeros((L,), jnp.float32)
        hi[pl.ds(i * L, L)] = jnp.zeros((L,), jnp.float32)
        return carry

    lax.fori_loop(0, NPR // L, z, 0)
    pltpu.sync_copy(src3.at[wid], idxs)
    pltpu.sync_copy(dst3.at[wid], idxd)

    ones = jnp.full((L,), 1.0, jnp.float32)

    def scat(j, carry):
        for k in range(C // L):
            plsc.addupdate_scatter(ho, [idxs[j, pl.ds(k * L, L)]], ones)
            plsc.addupdate_scatter(hi, [idxd[j, pl.ds(k * L, L)]], ones)
        return carry

    lax.fori_loop(0, CH, scat, 0)

    for h_v, out in ((ho, dego), (hi, degi)):
        pltpu.sync_copy(h_v, sh.at[s])
        plsc.subcore_barrier()
        for r in range(NS):
            pltpu.sync_copy(sh.at[r, pl.ds(s * RPT, RPT)], tmp.at[r])

        def red(g, carry):
            acc = tmp[0, pl.ds(g * L, L)]
            for r in range(1, NS):
                acc = acc + tmp[r, pl.ds(g * L, L)]
            res[pl.ds(g * L, L)] = acc
            return carry

        lax.fori_loop(0, RPT // L, red, 0)
        pltpu.sync_copy(res, out.at[c, pl.ds(s * RPT, RPT)])
        plsc.subcore_barrier()


@functools.partial(
    pl.kernel,
    out_type=jax.ShapeDtypeStruct((NC, NPR, D), jnp.float32),
    mesh=_mesh,
    scratch_types=[
        pltpu.VMEM((CP, C), jnp.int32),
        pltpu.VMEM((CP, C), jnp.int32),
        pltpu.VMEM((C, D), jnp.float32),
        pltpu.VMEM((C, D), jnp.float32),
        pltpu.VMEM_SHARED((NPR, D), jnp.float32),
        pltpu.SemaphoreType.DMA,
        pltpu.SemaphoreType.DMA,
    ],
)
def _agg_kernel(h_hbm, src3, dst3, aggp, idxs, idxd, vb0, vb1, sh_a,
                sem0, sem1):
    c = lax.axis_index("c")
    s = lax.axis_index("s")
    wid = c * NS + s

    def zfill(i, carry):
        for g in range(D // L):
            vb0[i, pl.ds(g * L, L)] = jnp.zeros((L,), jnp.float32)
        return carry

    lax.fori_loop(0, C, zfill, 0)
    for k in range(RPT // C):
        pltpu.sync_copy(vb0, sh_a.at[pl.ds(s * RPT + k * C, C)])
    plsc.subcore_barrier()

    for p in range(PH):
        pltpu.sync_copy(src3.at[wid, pl.ds(p * CP, CP)], idxs)
        pltpu.sync_copy(dst3.at[wid, pl.ds(p * CP, CP)], idxd)

        pltpu.async_copy(h_hbm.at[idxs.at[0]], vb0, sem0)
        pltpu.async_copy(h_hbm.at[idxs.at[1]], vb1, sem1)

        def step(jj, carry):
            j = jj * 2
            pltpu.make_async_copy(h_hbm.at[idxs.at[j]], vb0, sem0).wait()
            pltpu.sync_copy(vb0, sh_a.at[idxd.at[j]], add=True)
            pltpu.async_copy(h_hbm.at[idxs.at[j + 2]], vb0, sem0)
            pltpu.make_async_copy(h_hbm.at[idxs.at[j + 1]], vb1, sem1).wait()
            pltpu.sync_copy(vb1, sh_a.at[idxd.at[j + 1]], add=True)
            pltpu.async_copy(h_hbm.at[idxs.at[j + 3]], vb1, sem1)
            return carry

        lax.fori_loop(0, CP // 2 - 1, step, 0)
        jt = CP - 2
        pltpu.make_async_copy(h_hbm.at[idxs.at[jt]], vb0, sem0).wait()
        pltpu.sync_copy(vb0, sh_a.at[idxd.at[jt]], add=True)
        pltpu.make_async_copy(h_hbm.at[idxs.at[jt + 1]], vb1, sem1).wait()
        pltpu.sync_copy(vb1, sh_a.at[idxd.at[jt + 1]], add=True)

    plsc.subcore_barrier()
    pltpu.sync_copy(sh_a.at[pl.ds(s * RPT, RPT)], aggp.at[c, pl.ds(s * RPT, RPT)])


def _scale_body(dego_ref, x_ref, h_ref):
    d = dego_ref[0] + dego_ref[1]
    nrm = lax.rsqrt(jnp.maximum(d, 1.0))
    h_ref[...] = x_ref[...] * nrm[:, None]


_scale = pl.pallas_call(
    _scale_body,
    out_shape=jax.ShapeDtypeStruct((NPR, D), jnp.float32),
)


def _out_body(degi_ref, agg_ref, x_ref, w_ref, b_ref, o_ref):
    d = degi_ref[0] + degi_ref[1]
    nrm = lax.rsqrt(jnp.maximum(d, 1.0))
    a = (agg_ref[0] + agg_ref[1]) * nrm[:, None]
    a = a[:N]
    o_ref[...] = (
        jnp.dot(a, w_ref[...], preferred_element_type=jnp.float32)
        + b_ref[...][None, :]
        + x_ref[...]
    )


_finish = pl.pallas_call(
    _out_body,
    out_shape=jax.ShapeDtypeStruct((N, D), jnp.float32),
)


def kernel(x, edge_index, W, b):
    src = edge_index[0].astype(jnp.int32)
    dst = edge_index[1].astype(jnp.int32)
    pad = jnp.full((EPAD - E,), N, jnp.int32)
    src3 = jnp.concatenate([src, pad]).reshape(NW, CH, C)
    dst3 = jnp.concatenate([dst, pad]).reshape(NW, CH, C)
    x_pad = jnp.concatenate([x, jnp.zeros((NPR - N, D), jnp.float32)])

    dego, degi = _deg_kernel(src3, dst3)
    h_pad = _scale(dego, x_pad)
    aggp = _agg_kernel(h_pad, src3, dst3)
    out = _finish(degi, aggp, x, W, b)
    return out
